# Initial kernel scaffold; baseline (speedup 1.0000x reference)
import jax
import jax.numpy as jnp
from jax.experimental import pallas as pl


def kernel(hidden_states, Wq, bq, Wk, bk, Wv, bv, Wo, bo, ln1_g, ln1_b, ln2_g, ln2_b, Wr, W_up, b_up, W_gate, b_gate, W_down, b_down):
    raise NotImplementedError("write your pallas kernel here")


# speedup vs baseline: 2.2508x; 2.2508x over previous
import functools

import jax
import jax.numpy as jnp
from jax.experimental import pallas as pl
from jax.experimental.pallas import tpu as pltpu
from jax.experimental.pallas import tpu_sc as plsc

B, S, D, H, FF, E, K = 1, 2048, 768, 12, 3072, 8, 2
DH = D // H

BLK = 128
PAD = S * K + E * BLK
NB = PAD // BLK
BSQ = 512
BQ = 512

_NC, _NS = 2, 16
_NW = _NC * _NS


def _qkv_body(x_ref, wq_ref, wk_ref, wv_ref, bq_ref, bk_ref, bv_ref,
              q_ref, k_ref, v_ref):
    x = x_ref[...]
    q = jnp.dot(x, wq_ref[...], preferred_element_type=jnp.float32)
    q_ref[...] = (q + bq_ref[...]) * 0.125
    k_ref[...] = jnp.dot(x, wk_ref[...], preferred_element_type=jnp.float32) + bk_ref[...]
    v_ref[...] = jnp.dot(x, wv_ref[...], preferred_element_type=jnp.float32) + bv_ref[...]


def _qkv_call(x, Wq, bq, Wk, bk, Wv, bv):
    f = jax.ShapeDtypeStruct((S, D), jnp.float32)
    wspec = pl.BlockSpec((D, D), lambda i: (0, 0))
    bspec = pl.BlockSpec((1, D), lambda i: (0, 0))
    xspec = pl.BlockSpec((BSQ, D), lambda i: (i, 0))
    return pl.pallas_call(
        _qkv_body,
        grid=(S // BSQ,),
        in_specs=[xspec, wspec, wspec, wspec, bspec, bspec, bspec],
        out_specs=[xspec, xspec, xspec],
        out_shape=[f, f, f],
    )(x, Wq, Wk, Wv, bq.reshape(1, D), bk.reshape(1, D), bv.reshape(1, D))


def _attn_one(q, k, v):
    s = jax.lax.dot_general(q, k, (((1,), (1,)), ((), ())),
                            preferred_element_type=jnp.float32)
    m = jnp.max(s, axis=-1, keepdims=True)
    p = jnp.exp(s - m)
    p = p / jnp.sum(p, axis=-1, keepdims=True)
    return jnp.dot(p, v, preferred_element_type=jnp.float32)


def _attn_body(q_ref, k_ref, v_ref, o_ref):
    q = q_ref[...]
    k = k_ref[...]
    v = v_ref[...]
    oa = _attn_one(q[:, :DH], k[:, :DH], v[:, :DH])
    ob = _attn_one(q[:, DH:], k[:, DH:], v[:, DH:])
    o_ref[...] = jnp.concatenate([oa, ob], axis=1)


def _attn_call(q, k, v):
    return pl.pallas_call(
        _attn_body,
        grid=(H // 2, S // BQ),
        in_specs=[
            pl.BlockSpec((BQ, 2 * DH), lambda h, i: (i, h)),
            pl.BlockSpec((S, 2 * DH), lambda h, i: (0, h)),
            pl.BlockSpec((S, 2 * DH), lambda h, i: (0, h)),
        ],
        out_specs=pl.BlockSpec((BQ, 2 * DH), lambda h, i: (i, h)),
        out_shape=jax.ShapeDtypeStruct((S, D), jnp.float32),
    )(q, k, v)


def _ln(y, g, b):
    mu = jnp.mean(y, axis=-1, keepdims=True)
    var = jnp.mean((y - mu) ** 2, axis=-1, keepdims=True)
    return (y - mu) / jnp.sqrt(var + 1e-12) * g + b


def _post_body(ctx_ref, x_ref, wo_ref, bo_ref, g1_ref, b1_ref, g2_ref, b2_ref,
               wr_ref, attn_ref, t_ref, logits_ref, w1_ref, w2_ref,
               i1_ref, i2_ref):
    y = jnp.dot(ctx_ref[...], wo_ref[...], preferred_element_type=jnp.float32)
    y = y + bo_ref[...] + x_ref[...]
    a = _ln(y, g1_ref[...], b1_ref[...])
    attn_ref[...] = a
    t = _ln(a, g2_ref[...], b2_ref[...])
    t_ref[...] = t
    logits = jnp.dot(t, wr_ref[...], preferred_element_type=jnp.float32)
    logits_ref[...] = logits
    lm = jnp.max(logits, axis=-1, keepdims=True)
    ex = jnp.exp(logits - lm)
    probs = ex / jnp.sum(ex, axis=-1, keepdims=True)
    iota = jax.lax.broadcasted_iota(jnp.int32, probs.shape, 1)
    p1 = jnp.max(probs, axis=-1, keepdims=True)
    i1 = jnp.min(jnp.where(probs == p1, iota, E), axis=-1, keepdims=True)
    masked = jnp.where(iota == i1, -1.0, probs)
    p2 = jnp.max(masked, axis=-1, keepdims=True)
    i2 = jnp.min(jnp.where(masked == p2, iota, E), axis=-1, keepdims=True)
    tot = p1 + p2
    w1_ref[...] = p1 / tot
    w2_ref[...] = p2 / tot
    i1_ref[...] = i1
    i2_ref[...] = i2


def _post_call(ctx, x, Wo, bo, g1, b1, g2, b2, Wr):
    row = pl.BlockSpec((BSQ, D), lambda i: (i, 0))
    vec = pl.BlockSpec((1, D), lambda i: (0, 0))
    return pl.pallas_call(
        _post_body,
        grid=(S // BSQ,),
        in_specs=[row, row,
                  pl.BlockSpec((D, D), lambda i: (0, 0)), vec,
                  vec, vec, vec, vec,
                  pl.BlockSpec((D, E), lambda i: (0, 0))],
        out_specs=[row, row,
                   pl.BlockSpec((BSQ, E), lambda i: (i, 0)),
                   pl.BlockSpec((BSQ, 1), lambda i: (i, 0)),
                   pl.BlockSpec((BSQ, 1), lambda i: (i, 0)),
                   pl.BlockSpec((BSQ, 1), lambda i: (i, 0)),
                   pl.BlockSpec((BSQ, 1), lambda i: (i, 0))],
        out_shape=[jax.ShapeDtypeStruct((S, D), jnp.float32),
                   jax.ShapeDtypeStruct((S, D), jnp.float32),
                   jax.ShapeDtypeStruct((S, E), jnp.float32),
                   jax.ShapeDtypeStruct((S, 1), jnp.float32),
                   jax.ShapeDtypeStruct((S, 1), jnp.float32),
                   jax.ShapeDtypeStruct((S, 1), jnp.int32),
                   jax.ShapeDtypeStruct((S, 1), jnp.int32)],
    )(ctx, x, Wo, bo.reshape(1, D), g1.reshape(1, D), b1.reshape(1, D),
      g2.reshape(1, D), b2.reshape(1, D), Wr)


def _gemm_body(be_ref, t_ref, wup_ref, bup_ref, wg_ref, bg_ref, wd_ref,
               bd_ref, o_ref):
    tb = t_ref[...].astype(jnp.bfloat16)
    up = jnp.dot(tb, wup_ref[0], preferred_element_type=jnp.float32) + bup_ref[0]
    gate = jnp.dot(tb, wg_ref[0], preferred_element_type=jnp.float32) + bg_ref[0]
    h = 0.5 * up * (1.0 + jax.lax.erf(up * (2.0 ** -0.5))) * gate
    o_ref[...] = (jnp.dot(h.astype(jnp.bfloat16), wd_ref[0],
                          preferred_element_type=jnp.float32) + bd_ref[0])


def _moe_gemm_call(block_expert, t_pad, Wup, bup, Wg, bg, Wd, bd):
    grid_spec = pltpu.PrefetchScalarGridSpec(
        num_scalar_prefetch=1,
        grid=(NB,),
        in_specs=[
            pl.BlockSpec((BLK, D), lambda b, be: (b, 0)),
            pl.BlockSpec((1, D, FF), lambda b, be: (be[b], 0, 0)),
            pl.BlockSpec((1, 1, FF), lambda b, be: (be[b], 0, 0)),
            pl.BlockSpec((1, D, FF), lambda b, be: (be[b], 0, 0)),
            pl.BlockSpec((1, 1, FF), lambda b, be: (be[b], 0, 0)),
            pl.BlockSpec((1, FF, D), lambda b, be: (be[b], 0, 0)),
            pl.BlockSpec((1, 1, D), lambda b, be: (be[b], 0, 0)),
        ],
        out_specs=pl.BlockSpec((BLK, D), lambda b, be: (b, 0)),
    )
    return pl.pallas_call(
        _gemm_body,
        grid_spec=grid_spec,
        out_shape=jax.ShapeDtypeStruct((PAD, D), jnp.float32),
    )(block_expert, t_pad, Wup, bup.reshape(E, 1, FF), Wg,
      bg.reshape(E, 1, FF), Wd, bd.reshape(E, 1, D))


def _combine_body(attn_ref, g_ref, w1_ref, w2_ref, o_ref):
    g = g_ref[...]
    o_ref[...] = (attn_ref[...]
                  + w1_ref[...] * g[:, :D]
                  + w2_ref[...] * g[:, D:])


def _combine_call(attn, g2d, w1, w2):
    row = pl.BlockSpec((BSQ, D), lambda i: (i, 0))
    return pl.pallas_call(
        _combine_body,
        grid=(S // BSQ,),
        in_specs=[row,
                  pl.BlockSpec((BSQ, 2 * D), lambda i: (i, 0)),
                  pl.BlockSpec((BSQ, 1), lambda i: (i, 0)),
                  pl.BlockSpec((BSQ, 1), lambda i: (i, 0))],
        out_specs=row,
        out_shape=jax.ShapeDtypeStruct((S, D), jnp.float32),
    )(attn, g2d, w1, w2)


def _gather_rows(table, idx):
    n_rows = idx.shape[0]
    d = table.shape[1]
    b_per_w = n_rows // _NW
    mesh = plsc.VectorSubcoreMesh(core_axis_name="c", subcore_axis_name="s")

    @functools.partial(
        pl.kernel, mesh=mesh,
        out_type=jax.ShapeDtypeStruct((n_rows, d), jnp.float32),
        scratch_types=[pltpu.VMEM((b_per_w,), jnp.int32),
                       pltpu.VMEM((b_per_w, d), jnp.float32),
                       pltpu.SemaphoreType.DMA],
    )
    def k(table_hbm, idx_hbm, out_hbm, idx_v, rows_v, sem):
        wid = jax.lax.axis_index("s") * _NC + jax.lax.axis_index("c")
        base = wid * b_per_w
        pltpu.sync_copy(idx_hbm.at[pl.ds(base, b_per_w)], idx_v)
        pltpu.async_copy(table_hbm.at[idx_v], rows_v, sem).wait()
        pltpu.sync_copy(rows_v, out_hbm.at[pl.ds(base, b_per_w)])

    return k(table, idx)


def kernel(hidden_states, Wq, bq, Wk, bk, Wv, bv, Wo, bo, ln1_g, ln1_b,
           ln2_g, ln2_b, Wr, W_up, b_up, W_gate, b_gate, W_down, b_down):
    x = hidden_states.reshape(S, D)
    q, k, v = _qkv_call(x, Wq, bq, Wk, bk, Wv, bv)
    ctx = _attn_call(q, k, v)
    attn_out, t, logits, w1, w2, i1, i2 = _post_call(
        ctx, x, Wo, bo, ln1_g, ln1_b, ln2_g, ln2_b, Wr)

    e_flat = jnp.concatenate([i1, i2], axis=1).reshape(S * K)
    onehot = (e_flat[:, None] == jnp.arange(E)[None, :]).astype(jnp.int32)
    csum = jnp.cumsum(onehot, axis=0)
    counts = csum[-1]
    rank = jnp.take_along_axis(csum, e_flat[:, None], axis=1)[:, 0] - 1
    padded = ((counts + BLK - 1) // BLK) * BLK
    pstart = jnp.concatenate([jnp.zeros((1,), jnp.int32),
                              jnp.cumsum(padded)[:-1].astype(jnp.int32)])
    dp = pstart[e_flat] + rank
    row_token = jnp.zeros((PAD,), jnp.int32).at[dp].set(
        jnp.arange(S * K, dtype=jnp.int32) // K)
    block_expert = (jnp.sum(
        (pstart[None, :] <= (jnp.arange(NB, dtype=jnp.int32) * BLK)[:, None]
         ).astype(jnp.int32), axis=1) - 1).astype(jnp.int32)

    t_pad = _gather_rows(t, row_token)

    moe_sorted = _moe_gemm_call(
        block_expert, t_pad,
        W_up.astype(jnp.bfloat16), b_up,
        W_gate.astype(jnp.bfloat16), b_gate,
        W_down.astype(jnp.bfloat16), b_down)

    g = _gather_rows(moe_sorted, dp)
    out = _combine_call(attn_out, g.reshape(S, K * D), w1, w2)
    return out.reshape(B, S, D), logits
